# baseline (speedup 1.0000x reference)
import functools

import jax
import jax.numpy as jnp
import numpy as np
from jax import lax
from jax.experimental import pallas as pl
from jax.experimental.pallas import tpu as pltpu
from jax.experimental.pallas import tpu_sc as plsc

VOCAB = 100000
EMB = 128
B = 4096
L = 200
NUM_CLASSES = 2
HIDDEN = 200

NC = 2
NS = 16
NW = NC * NS
ROWS_W = B // NW
CHUNK = L // 2
CHUNKS_W = ROWS_W * 2
NBUF = 4
EMB2 = EMB // 2

_PERM = np.concatenate([
    np.concatenate([32 * q + 2 * np.arange(16), 32 * q + 2 * np.arange(16) + 1])
    for q in range(4)
])


def _sc_body(table_hbm, ids_hbm, out_hbm, idx_v, bufs, pooled_v, sems):
    cid = lax.axis_index("c")
    sid = lax.axis_index("s")
    wid = sid * NC + cid
    rbase = wid * CHUNKS_W
    obase = wid * ROWS_W

    pltpu.sync_copy(ids_hbm.at[pl.ds(rbase, CHUNKS_W)], idx_v)

    def gather(c, k):
        return pltpu.make_async_copy(table_hbm.at[idx_v.at[c]], bufs.at[k],
                                     sems.at[k])

    for k in range(NBUF):
        gather(k, k).start()

    def outer(i, _):
        for half in range(2):
            acc = tuple(jnp.zeros((16,), jnp.float32) for _ in range(8))
            for k2 in range(2):
                k = half * 2 + k2
                c = 4 * i + k
                gather(c, k).wait()

                def red(j, carry, k=k):
                    out = []
                    for q in range(4):
                        x = bufs[k, j, pl.ds(q * 16, 16)]
                        lo = lax.bitcast_convert_type(
                            lax.shift_left(x, 16), jnp.float32)
                        hi = lax.bitcast_convert_type(
                            lax.bitwise_and(x, jnp.int32(-65536)),
                            jnp.float32)
                        out.append(carry[2 * q] + lo)
                        out.append(carry[2 * q + 1] + hi)
                    return tuple(out)

                acc = lax.fori_loop(0, CHUNK, red, acc)

                @pl.when(c + NBUF < CHUNKS_W)
                def _(c=c, k=k):
                    gather(c + NBUF, k).start()

            row = 2 * i + half
            inv_l = jnp.float32(1.0 / L)
            for q in range(4):
                pooled_v[row, pl.ds(q * 32, 16)] = acc[2 * q] * inv_l
                pooled_v[row, pl.ds(q * 32 + 16, 16)] = acc[2 * q + 1] * inv_l
        return 0

    lax.fori_loop(0, ROWS_W // 2, outer, 0)
    pltpu.sync_copy(pooled_v, out_hbm.at[pl.ds(obase, ROWS_W)])


def _sc_pool(table_i32, ids2):
    mesh = plsc.VectorSubcoreMesh(core_axis_name="c", subcore_axis_name="s",
                                  num_cores=NC, num_subcores=NS)
    return pl.kernel(
        _sc_body,
        out_type=jax.ShapeDtypeStruct((B, EMB), jnp.float32),
        mesh=mesh,
        scratch_types=[
            pltpu.VMEM((CHUNKS_W, CHUNK), jnp.int32),
            pltpu.VMEM((NBUF, CHUNK, EMB2), jnp.int32),
            pltpu.VMEM((ROWS_W, EMB), jnp.float32),
            pltpu.SemaphoreType.DMA((NBUF,)),
        ],
        compiler_params=pltpu.CompilerParams(use_tc_tiling_on_sc=False),
    )(table_i32, ids2)


def _tc_body(x_ref, w1_ref, b1_ref, w2_ref, b2_ref, lab_ref,
             logits_ref, loss_ref):
    x = x_ref[...]
    h = jnp.maximum(
        jnp.dot(x, w1_ref[...], preferred_element_type=jnp.float32)
        + b1_ref[...], 0.0)
    lg = (jnp.dot(h, w2_ref[...], preferred_element_type=jnp.float32)
          + b2_ref[...])
    logits_ref[...] = lg
    m = jnp.max(lg, axis=1, keepdims=True)
    se = jnp.sum(jnp.exp(lg - m), axis=1, keepdims=True)
    lse = m + jnp.log(se)
    col = lax.broadcasted_iota(jnp.int32, (B, NUM_CLASSES), 1)
    picked = jnp.sum(jnp.where(col == lab_ref[...], lg, 0.0), axis=1,
                     keepdims=True)
    loss_ref[0, 0] = jnp.sum(lse - picked) / jnp.float32(B)


def _tc_mlp(pooled, W1, b1, W2, b2, labels2d):
    return pl.pallas_call(
        _tc_body,
        out_shape=(
            jax.ShapeDtypeStruct((B, NUM_CLASSES), jnp.float32),
            jax.ShapeDtypeStruct((1, 1), jnp.float32),
        ),
        out_specs=(
            pl.BlockSpec(memory_space=pltpu.VMEM),
            pl.BlockSpec(memory_space=pltpu.SMEM),
        ),
    )(pooled, W1, b1, W2, b2, labels2d)


def kernel(input_ids, labels, emb_table, W1, b1, W2, b2):
    ids2 = input_ids.astype(jnp.int32).reshape(B * 2, CHUNK)
    t16 = emb_table.astype(jnp.bfloat16)
    tbl_i32 = lax.bitcast_convert_type(
        t16.reshape(VOCAB, EMB2, 2), jnp.int32)
    pooled = _sc_pool(tbl_i32, ids2)
    W1p = W1[_PERM, :]
    logits, loss = _tc_mlp(pooled, W1p, b1.reshape(1, HIDDEN), W2,
                           b2.reshape(1, NUM_CLASSES),
                           labels.astype(jnp.int32).reshape(B, 1))
    return (logits, loss.reshape(()))


# speedup vs baseline: 2.3685x; 2.3685x over previous
import functools

import jax
import jax.numpy as jnp
import numpy as np
from jax import lax
from jax.experimental import pallas as pl
from jax.experimental.pallas import tpu as pltpu
from jax.experimental.pallas import tpu_sc as plsc

VOCAB = 100000
EMB = 128
B = 4096
L = 200
NUM_CLASSES = 2
HIDDEN = 200

NC = 2
NS = 16
NW = NC * NS
ROWS_W = B // NW
CHUNK = L // 2
CHUNKS_W = ROWS_W * 2
NBUF = 4
EMB2 = EMB // 2

_PERM = np.concatenate([
    np.concatenate([16 * q + np.arange(16), 64 + 16 * q + np.arange(16)])
    for q in range(4)
])


def _sc_body(table_hbm, ids_hbm, out_hbm, idx_v, bufs, pooled_v, sems):
    cid = lax.axis_index("c")
    sid = lax.axis_index("s")
    wid = sid * NC + cid
    rbase = wid * CHUNKS_W
    obase = wid * ROWS_W

    pltpu.sync_copy(ids_hbm.at[pl.ds(rbase, CHUNKS_W)], idx_v)

    def gather(c, k):
        return pltpu.make_async_copy(table_hbm.at[idx_v.at[c]], bufs.at[k],
                                     sems.at[k])

    for k in range(NBUF):
        gather(k, k).start()

    def outer(i, _):
        for half in range(2):
            acc = tuple(jnp.zeros((16,), jnp.float32) for _ in range(8))
            for k2 in range(2):
                k = half * 2 + k2
                c = 4 * i + k
                gather(c, k).wait()

                def red(j, carry, k=k):
                    out = []
                    for q in range(4):
                        x = bufs[k, j, pl.ds(q * 16, 16)]
                        lo = lax.bitcast_convert_type(
                            lax.shift_left(x, 16), jnp.float32)
                        hi = lax.bitcast_convert_type(
                            lax.bitwise_and(x, jnp.int32(-65536)),
                            jnp.float32)
                        out.append(carry[2 * q] + lo)
                        out.append(carry[2 * q + 1] + hi)
                    return tuple(out)

                acc = lax.fori_loop(0, CHUNK, red, acc)

                @pl.when(c + NBUF < CHUNKS_W)
                def _(c=c, k=k):
                    gather(c + NBUF, k).start()

            row = 2 * i + half
            inv_l = jnp.float32(1.0 / L)
            for q in range(4):
                pooled_v[row, pl.ds(q * 32, 16)] = acc[2 * q] * inv_l
                pooled_v[row, pl.ds(q * 32 + 16, 16)] = acc[2 * q + 1] * inv_l
        return 0

    lax.fori_loop(0, ROWS_W // 2, outer, 0)
    pltpu.sync_copy(pooled_v, out_hbm.at[pl.ds(obase, ROWS_W)])


def _sc_pool(table_i32, ids2):
    mesh = plsc.VectorSubcoreMesh(core_axis_name="c", subcore_axis_name="s",
                                  num_cores=NC, num_subcores=NS)
    return pl.kernel(
        _sc_body,
        out_type=jax.ShapeDtypeStruct((B, EMB), jnp.float32),
        mesh=mesh,
        scratch_types=[
            pltpu.VMEM((CHUNKS_W, CHUNK), jnp.int32),
            pltpu.VMEM((NBUF, CHUNK, EMB2), jnp.int32),
            pltpu.VMEM((ROWS_W, EMB), jnp.float32),
            pltpu.SemaphoreType.DMA((NBUF,)),
        ],
        compiler_params=pltpu.CompilerParams(use_tc_tiling_on_sc=False),
    )(table_i32, ids2)


def _tc_body(x_ref, w1_ref, b1_ref, w2_ref, b2_ref, lab_ref,
             logits_ref, loss_ref):
    x = x_ref[...]
    h = jnp.maximum(
        jnp.dot(x, w1_ref[...], preferred_element_type=jnp.float32)
        + b1_ref[...], 0.0)
    lg = (jnp.dot(h, w2_ref[...], preferred_element_type=jnp.float32)
          + b2_ref[...])
    logits_ref[...] = lg
    m = jnp.max(lg, axis=1, keepdims=True)
    se = jnp.sum(jnp.exp(lg - m), axis=1, keepdims=True)
    lse = m + jnp.log(se)
    col = lax.broadcasted_iota(jnp.int32, (B, NUM_CLASSES), 1)
    picked = jnp.sum(jnp.where(col == lab_ref[...], lg, 0.0), axis=1,
                     keepdims=True)
    loss_ref[0, 0] = jnp.sum(lse - picked) / jnp.float32(B)


def _tc_mlp(pooled, W1, b1, W2, b2, labels2d):
    return pl.pallas_call(
        _tc_body,
        out_shape=(
            jax.ShapeDtypeStruct((B, NUM_CLASSES), jnp.float32),
            jax.ShapeDtypeStruct((1, 1), jnp.float32),
        ),
        out_specs=(
            pl.BlockSpec(memory_space=pltpu.VMEM),
            pl.BlockSpec(memory_space=pltpu.SMEM),
        ),
    )(pooled, W1, b1, W2, b2, labels2d)


def kernel(input_ids, labels, emb_table, W1, b1, W2, b2):
    ids2 = input_ids.astype(jnp.int32).reshape(B * 2, CHUNK)
    bits = lax.bitcast_convert_type(emb_table, jnp.int32)
    rn = lax.shift_right_logical(bits, 16) & 1
    bf = bits + 0x7FFF + rn
    lo = lax.shift_right_logical(bf[:, :EMB2], 16)
    hi = lax.bitwise_and(bf[:, EMB2:], jnp.int32(-65536))
    tbl_i32 = lax.bitwise_or(lo, hi)
    pooled = _sc_pool(tbl_i32, ids2)
    W1p = W1[_PERM, :]
    logits, loss = _tc_mlp(pooled, W1p, b1.reshape(1, HIDDEN), W2,
                           b2.reshape(1, NUM_CLASSES),
                           labels.astype(jnp.int32).reshape(B, 1))
    return (logits, loss.reshape(()))


# speedup vs baseline: 3.4095x; 1.4395x over previous
import functools

import jax
import jax.numpy as jnp
from jax import lax
from jax.experimental import pallas as pl
from jax.experimental.pallas import tpu as pltpu
from jax.experimental.pallas import tpu_sc as plsc

VOCAB = 100000
EMB = 128
B = 4096
L = 200
NUM_CLASSES = 2
HIDDEN = 200

NC = 2
NS = 16
NW = NC * NS
ROWS_W = B // NW
CHUNK = L // 2
CHUNKS_W = ROWS_W * 2
NBUF = 4


def _sc_body(table_hbm, ids_hbm, out_hbm, idx_v, bufs, pooled_v, sems):
    cid = lax.axis_index("c")
    sid = lax.axis_index("s")
    wid = sid * NC + cid
    rbase = wid * CHUNKS_W
    obase = wid * ROWS_W

    pltpu.sync_copy(ids_hbm.at[pl.ds(rbase, CHUNKS_W)], idx_v)

    def gather(c, k):
        return pltpu.make_async_copy(table_hbm.at[idx_v.at[c]], bufs.at[k],
                                     sems.at[k])

    for k in range(NBUF):
        gather(k, k).start()

    def outer(i, _):
        for half in range(2):
            acc = tuple(jnp.zeros((16,), jnp.float32) for _ in range(8))
            for k2 in range(2):
                k = half * 2 + k2
                c = 4 * i + k
                gather(c, k).wait()

                def red(j, carry, k=k):
                    return tuple(carry[d] + bufs[k, j, pl.ds(d * 16, 16)]
                                 for d in range(8))

                acc = lax.fori_loop(0, CHUNK, red, acc)

                @pl.when(c + NBUF < CHUNKS_W)
                def _(c=c, k=k):
                    gather(c + NBUF, k).start()

            row = 2 * i + half
            inv_l = jnp.float32(1.0 / L)
            for d in range(8):
                pooled_v[row, pl.ds(d * 16, 16)] = acc[d] * inv_l
        return 0

    lax.fori_loop(0, ROWS_W // 2, outer, 0)
    pltpu.sync_copy(pooled_v, out_hbm.at[pl.ds(obase, ROWS_W)])


@functools.partial(jax.jit, static_argnums=())
def _sc_pool(emb_table, ids2):
    mesh = plsc.VectorSubcoreMesh(core_axis_name="c", subcore_axis_name="s",
                                  num_cores=NC, num_subcores=NS)
    return pl.kernel(
        _sc_body,
        out_type=jax.ShapeDtypeStruct((B, EMB), jnp.float32),
        mesh=mesh,
        scratch_types=[
            pltpu.VMEM((CHUNKS_W, CHUNK), jnp.int32),
            pltpu.VMEM((NBUF, CHUNK, EMB), jnp.float32),
            pltpu.VMEM((ROWS_W, EMB), jnp.float32),
            pltpu.SemaphoreType.DMA((NBUF,)),
        ],
    )(emb_table, ids2)


def _tc_body(x_ref, w1_ref, b1_ref, w2_ref, b2_ref, lab_ref,
             logits_ref, loss_ref):
    x = x_ref[...]
    h = jnp.maximum(
        jnp.dot(x, w1_ref[...], preferred_element_type=jnp.float32)
        + b1_ref[...], 0.0)
    lg = (jnp.dot(h, w2_ref[...], preferred_element_type=jnp.float32)
          + b2_ref[...])
    logits_ref[...] = lg
    m = jnp.max(lg, axis=1, keepdims=True)
    se = jnp.sum(jnp.exp(lg - m), axis=1, keepdims=True)
    lse = m + jnp.log(se)
    col = lax.broadcasted_iota(jnp.int32, (B, NUM_CLASSES), 1)
    picked = jnp.sum(jnp.where(col == lab_ref[...], lg, 0.0), axis=1,
                     keepdims=True)
    loss_ref[0, 0] = jnp.sum(lse - picked) / jnp.float32(B)


def _tc_mlp(pooled, W1, b1, W2, b2, labels2d):
    return pl.pallas_call(
        _tc_body,
        out_shape=(
            jax.ShapeDtypeStruct((B, NUM_CLASSES), jnp.float32),
            jax.ShapeDtypeStruct((1, 1), jnp.float32),
        ),
        out_specs=(
            pl.BlockSpec(memory_space=pltpu.VMEM),
            pl.BlockSpec(memory_space=pltpu.SMEM),
        ),
    )(pooled, W1, b1, W2, b2, labels2d)


def kernel(input_ids, labels, emb_table, W1, b1, W2, b2):
    ids2 = input_ids.astype(jnp.int32).reshape(B * 2, CHUNK)
    pooled = _sc_pool(emb_table, ids2)
    logits, loss = _tc_mlp(pooled, W1, b1.reshape(1, HIDDEN), W2,
                           b2.reshape(1, NUM_CLASSES),
                           labels.astype(jnp.int32).reshape(B, 1))
    return (logits, loss.reshape(()))
